# baseline (speedup 1.0000x reference)
import jax
import jax.numpy as jnp
from jax import lax
from jax.experimental import pallas as pl
from jax.experimental.pallas import tpu as pltpu
from jax.experimental.pallas import tpu_sc as plsc

N_NODES = 10000
N_EDGES = 320000
F = 128
L = 16
FT = 4

NC = 2
NS = 16
NW = NC * NS

NPH = 10
PHE = N_EDGES // NPH
UNROLL = 4
XWORDS = N_NODES * FT


def _sc_body(xt_hbm, packed_hbm, out_hbm, x_v, acc_v, idx_v):
    c = lax.axis_index("c")
    s = lax.axis_index("s")
    t = c * NS + s
    pltpu.sync_copy(xt_hbm.at[t, 0], x_v)

    def zstep(i, carry):
        acc_v[pl.ds(i * L, L)] = jnp.zeros((L,), jnp.float32)
        return carry

    lax.fori_loop(0, XWORDS // L, zstep, 0)

    for ph in range(NPH):
        pltpu.sync_copy(packed_hbm.at[ph, 0], idx_v)

        def estep(k, carry):
            for m in range(UNROLL):
                p = idx_v[pl.ds(k * (L * UNROLL) + m * L, L)]
                s4 = (p & 0xFFFF) << 2
                d4 = (p >> 16) << 2
                for f in range(FT):
                    v = plsc.load_gather(x_v, [s4 + f])
                    plsc.addupdate_scatter(acc_v, [d4 + f], v)
            return carry

        lax.fori_loop(0, PHE // (L * UNROLL), estep, 0)

    pltpu.sync_copy(acc_v, out_hbm.at[t, 0])


@jax.jit
def _sc_scatter(xt, packed):
    mesh = plsc.VectorSubcoreMesh(core_axis_name="c", subcore_axis_name="s",
                                  num_cores=NC, num_subcores=NS)
    return pl.kernel(
        _sc_body,
        out_type=jax.ShapeDtypeStruct((NW, 1, XWORDS), jnp.float32),
        mesh=mesh,
        compiler_params=pltpu.CompilerParams(needs_layout_passes=False),
        scratch_types=[
            pltpu.VMEM((XWORDS,), jnp.float32),
            pltpu.VMEM((XWORDS,), jnp.float32),
            pltpu.VMEM((PHE,), jnp.int32),
        ],
    )(xt, packed)


def _tc_body(h_ref, w_ref, b_ref, g_ref, be_ref, out_ref):
    y = lax.dot_general(h_ref[...], w_ref[...], (((1,), (1,)), ((), ())),
                        preferred_element_type=jnp.float32,
                        precision=lax.Precision.HIGHEST)
    y = jnp.maximum(y + b_ref[...], 0.0)
    mean = jnp.mean(y, axis=0, keepdims=True)
    var = jnp.mean(jnp.square(y - mean), axis=0, keepdims=True)
    out_ref[...] = (y - mean) * lax.rsqrt(var + 1e-5) * g_ref[...] + be_ref[...]


@jax.jit
def _tc_finish(h, W, b, gamma, beta):
    return pl.pallas_call(
        _tc_body,
        out_shape=jax.ShapeDtypeStruct((N_NODES, F), jnp.float32),
    )(h, W, b.reshape(1, F), gamma.reshape(1, F), beta.reshape(1, F))


def kernel(x, edge_index, W, b, gamma, beta):
    xt = x.reshape(N_NODES, NW, FT).transpose(1, 0, 2).reshape(NW, 1, XWORDS)
    src = edge_index[0].astype(jnp.int32)
    dst = edge_index[1].astype(jnp.int32)
    packed = (src | (dst << 16)).reshape(NPH, 1, PHE)
    out = _sc_scatter(xt, packed)
    h = out.reshape(NW, N_NODES, FT).transpose(1, 0, 2).reshape(N_NODES, F)
    return _tc_finish(h, W, b, gamma, beta)


# speedup vs baseline: 1.9705x; 1.9705x over previous
import jax
import jax.numpy as jnp
from jax import lax
from jax.experimental import pallas as pl
from jax.experimental.pallas import tpu as pltpu
from jax.experimental.pallas import tpu_sc as plsc

N_NODES = 10000
N_EDGES = 320000
F = 128
L = 16
FT = 4

NC = 2
NS = 16
NW = NC * NS

NPH = 10
PHE = N_EDGES // NPH
UNROLL = 4
XWORDS = N_NODES * FT


def _sc_body(xt_hbm, packed_hbm, out_hbm, x_v, acc_v, idx_v):
    c = lax.axis_index("c")
    s = lax.axis_index("s")
    t = c * NS + s
    pltpu.sync_copy(xt_hbm.at[t, 0], x_v)

    def zstep(i, carry):
        acc_v[pl.ds(i * L, L)] = jnp.zeros((L,), jnp.float32)
        return carry

    lax.fori_loop(0, XWORDS // L, zstep, 0)

    for ph in range(NPH):
        pltpu.sync_copy(packed_hbm.at[ph, 0], idx_v)

        @plsc.parallel_loop(0, PHE // L, unroll=UNROLL)
        def _(k):
            p = idx_v[pl.ds(k * L, L)]
            s4 = (p & 0xFFFF) << 2
            d4 = (p >> 16) << 2
            for f in range(FT):
                v = plsc.load_gather(x_v, [s4 + f])
                plsc.addupdate_scatter(acc_v, [d4 + f], v)

    pltpu.sync_copy(acc_v, out_hbm.at[t, 0])


@jax.jit
def _sc_scatter(xt, packed):
    mesh = plsc.VectorSubcoreMesh(core_axis_name="c", subcore_axis_name="s",
                                  num_cores=NC, num_subcores=NS)
    return pl.kernel(
        _sc_body,
        out_type=jax.ShapeDtypeStruct((NW, 1, XWORDS), jnp.float32),
        mesh=mesh,
        compiler_params=pltpu.CompilerParams(needs_layout_passes=False),
        scratch_types=[
            pltpu.VMEM((XWORDS,), jnp.float32),
            pltpu.VMEM((XWORDS,), jnp.float32),
            pltpu.VMEM((PHE,), jnp.int32),
        ],
    )(xt, packed)


def _tc_body(h_ref, w_ref, b_ref, g_ref, be_ref, out_ref):
    y = lax.dot_general(h_ref[...], w_ref[...], (((1,), (1,)), ((), ())),
                        preferred_element_type=jnp.float32,
                        precision=lax.Precision.HIGHEST)
    y = jnp.maximum(y + b_ref[...], 0.0)
    mean = jnp.mean(y, axis=0, keepdims=True)
    var = jnp.mean(jnp.square(y - mean), axis=0, keepdims=True)
    out_ref[...] = (y - mean) * lax.rsqrt(var + 1e-5) * g_ref[...] + be_ref[...]


@jax.jit
def _tc_finish(h, W, b, gamma, beta):
    return pl.pallas_call(
        _tc_body,
        out_shape=jax.ShapeDtypeStruct((N_NODES, F), jnp.float32),
    )(h, W, b.reshape(1, F), gamma.reshape(1, F), beta.reshape(1, F))


def kernel(x, edge_index, W, b, gamma, beta):
    xt = x.reshape(N_NODES, NW, FT).transpose(1, 0, 2).reshape(NW, 1, XWORDS)
    src = edge_index[0].astype(jnp.int32)
    dst = edge_index[1].astype(jnp.int32)
    packed = (src | (dst << 16)).reshape(NPH, 1, PHE)
    out = _sc_scatter(xt, packed)
    h = out.reshape(NW, N_NODES, FT).transpose(1, 0, 2).reshape(N_NODES, F)
    return _tc_finish(h, W, b, gamma, beta)
